# Initial kernel scaffold; baseline (speedup 1.0000x reference)
import jax
import jax.numpy as jnp
from jax.experimental import pallas as pl


def kernel(h, edge_index):
    raise NotImplementedError("write your pallas kernel here")


# speedup vs baseline: 1.2696x; 1.2696x over previous
import functools

import jax
import jax.numpy as jnp
from jax import lax
from jax.experimental import pallas as pl
from jax.experimental.pallas import tpu as pltpu
from jax.experimental.pallas import tpu_sc as plsc

N_NODES = 10000
D_FEAT = 256
L = 16
NC, NS = 2, 16
NW = NC * NS
K = 128


@functools.cache
def _build(E_pad):
    per_w = E_pad // NW
    n_chunks = per_w // K
    n_groups = K // L
    mesh = plsc.VectorSubcoreMesh(core_axis_name="c", subcore_axis_name="s")

    @functools.partial(
        pl.kernel,
        mesh=mesh,
        out_type=jax.ShapeDtypeStruct((E_pad,), jnp.float32),
        scratch_types=[
            pltpu.VMEM((K,), jnp.int32),
            pltpu.VMEM((K,), jnp.int32),
            pltpu.VMEM((K, D_FEAT), jnp.float32),
            pltpu.VMEM((K, D_FEAT), jnp.float32),
            pltpu.VMEM((per_w,), jnp.float32),
            pltpu.SemaphoreType.DMA,
            pltpu.SemaphoreType.DMA,
        ],
    )
    def edge_dot(h_hbm, src_hbm, dst_hbm, out_hbm,
                 idx_s, idx_d, rows_s, rows_d, out_v, sem_s, sem_d):
        wid = lax.axis_index("s") * NC + lax.axis_index("c")
        w_base = wid * per_w
        lanes = lax.iota(jnp.int32, L)

        def chunk_body(c, _):
            base = w_base + c * K
            pltpu.sync_copy(src_hbm.at[pl.ds(base, K)], idx_s)
            pltpu.sync_copy(dst_hbm.at[pl.ds(base, K)], idx_d)
            cp_s = pltpu.async_copy(h_hbm.at[idx_s], rows_s, sem_s)
            cp_d = pltpu.async_copy(h_hbm.at[idx_d], rows_d, sem_d)
            cp_s.wait()
            cp_d.wait()

            def group_body(g, _):
                score = jnp.zeros((L,), jnp.float32)
                for e in range(L):
                    row = g * L + e
                    acc = rows_s[row, pl.ds(0, L)] * rows_d[row, pl.ds(0, L)]
                    for j in range(1, D_FEAT // L):
                        acc = acc + (rows_s[row, pl.ds(j * L, L)]
                                     * rows_d[row, pl.ds(j * L, L)])
                    r = acc[0]
                    for j in range(1, L):
                        r = r + acc[j]
                    score = jnp.where(lanes == e, r, score)
                out_v[pl.ds(c * K + g * L, L)] = score
                return 0

            lax.fori_loop(0, n_groups, group_body, 0, unroll=False)
            return 0

        lax.fori_loop(0, n_chunks, chunk_body, 0, unroll=False)

        def sig_body(v, _):
            s = out_v[pl.ds(v * L, L)]
            out_v[pl.ds(v * L, L)] = 1.0 / (1.0 + jnp.exp(-s))
            return 0

        lax.fori_loop(0, per_w // L, sig_body, 0, unroll=False)
        pltpu.sync_copy(out_v, out_hbm.at[pl.ds(w_base, per_w)])

    return edge_dot


def kernel(h, edge_index):
    src = edge_index[0].astype(jnp.int32)
    dst = edge_index[1].astype(jnp.int32)
    e = src.shape[0]
    e_pad = ((e + NW * K - 1) // (NW * K)) * (NW * K)
    pad = e_pad - e
    if pad:
        src = jnp.concatenate([src, jnp.zeros((pad,), jnp.int32)])
        dst = jnp.concatenate([dst, jnp.zeros((pad,), jnp.int32)])
    out = _build(e_pad)(h, src, dst)
    return out[:e]


# speedup vs baseline: 1.6336x; 1.2867x over previous
import functools

import jax
import jax.numpy as jnp
from jax import lax
from jax.experimental import pallas as pl
from jax.experimental.pallas import tpu as pltpu
from jax.experimental.pallas import tpu_sc as plsc

N_NODES = 10000
D_FEAT = 256
L = 16
NC, NS = 2, 16
NW = NC * NS
K = 64


@functools.cache
def _build(E_pad):
    per_w = E_pad // NW
    n_chunks = per_w // K
    n_groups = K // L
    assert n_chunks % 2 == 0
    mesh = plsc.VectorSubcoreMesh(core_axis_name="c", subcore_axis_name="s")

    @functools.partial(
        pl.kernel,
        mesh=mesh,
        out_type=jax.ShapeDtypeStruct((E_pad,), jnp.float32),
        scratch_types=[
            pltpu.VMEM((per_w,), jnp.int32),
            pltpu.VMEM((per_w,), jnp.int32),
            pltpu.VMEM((K, D_FEAT), jnp.float32),
            pltpu.VMEM((K, D_FEAT), jnp.float32),
            pltpu.VMEM((K, D_FEAT), jnp.float32),
            pltpu.VMEM((K, D_FEAT), jnp.float32),
            pltpu.VMEM((per_w,), jnp.float32),
            pltpu.SemaphoreType.DMA,
            pltpu.SemaphoreType.DMA,
            pltpu.SemaphoreType.DMA,
            pltpu.SemaphoreType.DMA,
        ],
    )
    def edge_dot(h_hbm, src_hbm, dst_hbm, out_hbm,
                 idx_s, idx_d, rows_s0, rows_s1, rows_d0, rows_d1,
                 out_v, sem_s0, sem_s1, sem_d0, sem_d1):
        wid = lax.axis_index("s") * NC + lax.axis_index("c")
        w_base = wid * per_w
        lanes = lax.iota(jnp.int32, L)
        rows_s = (rows_s0, rows_s1)
        rows_d = (rows_d0, rows_d1)
        sem_s = (sem_s0, sem_s1)
        sem_d = (sem_d0, sem_d1)

        def start_gathers(c, b):
            pltpu.async_copy(
                h_hbm.at[idx_s.at[pl.ds(c * K, K)]], rows_s[b], sem_s[b])
            pltpu.async_copy(
                h_hbm.at[idx_d.at[pl.ds(c * K, K)]], rows_d[b], sem_d[b])

        def wait_gathers(b):
            pltpu.make_async_copy(
                h_hbm.at[pl.ds(0, K)], rows_s[b], sem_s[b]).wait()
            pltpu.make_async_copy(
                h_hbm.at[pl.ds(0, K)], rows_d[b], sem_d[b]).wait()

        pltpu.sync_copy(src_hbm.at[pl.ds(w_base, per_w)], idx_s)
        pltpu.sync_copy(dst_hbm.at[pl.ds(w_base, per_w)], idx_d)
        start_gathers(0, 0)

        def compute_chunk(c, b):
            rs, rd = rows_s[b], rows_d[b]

            def group_body(g, _):
                score = jnp.zeros((L,), jnp.float32)
                for e in range(L):
                    row = g * L + e
                    acc = rs[row, pl.ds(0, L)] * rd[row, pl.ds(0, L)]
                    for j in range(1, D_FEAT // L):
                        acc = acc + (rs[row, pl.ds(j * L, L)]
                                     * rd[row, pl.ds(j * L, L)])
                    r = acc[0]
                    for j in range(1, L):
                        r = r + acc[j]
                    score = jnp.where(lanes == e, r, score)
                out_v[pl.ds(c * K + g * L, L)] = score
                return 0

            lax.fori_loop(0, n_groups, group_body, 0, unroll=False)

        def pipe_body(cc, _):
            for b in range(2):
                c = cc * 2 + b

                @pl.when(c + 1 < n_chunks)
                def _():
                    start_gathers(c + 1, 1 - b)

                wait_gathers(b)
                compute_chunk(c, b)
            return 0

        lax.fori_loop(0, n_chunks // 2, pipe_body, 0, unroll=False)

        def sig_body(v, _):
            s = out_v[pl.ds(v * L, L)]
            out_v[pl.ds(v * L, L)] = 1.0 / (1.0 + jnp.exp(-s))
            return 0

        lax.fori_loop(0, per_w // L, sig_body, 0, unroll=False)
        pltpu.sync_copy(out_v, out_hbm.at[pl.ds(w_base, per_w)])

    return edge_dot


def kernel(h, edge_index):
    src = edge_index[0].astype(jnp.int32)
    dst = edge_index[1].astype(jnp.int32)
    e = src.shape[0]
    blk = NW * K * 2
    e_pad = ((e + blk - 1) // blk) * blk
    pad = e_pad - e
    if pad:
        src = jnp.concatenate([src, jnp.zeros((pad,), jnp.int32)])
        dst = jnp.concatenate([dst, jnp.zeros((pad,), jnp.int32)])
    out = _build(e_pad)(h, src, dst)
    return out[:e]
